# Initial kernel scaffold; baseline (speedup 1.0000x reference)
import jax
import jax.numpy as jnp
from jax.experimental import pallas as pl


def kernel(rank, X_in, ones):
    raise NotImplementedError("write your pallas kernel here")


# speedup vs baseline: 84.8784x; 84.8784x over previous
import jax
import jax.numpy as jnp
from jax.experimental import pallas as pl

DEPTH = 32
H = 512
W = 512
HB = 128


def _one_hot_kernel(x_ref, o_ref):
    x = x_ref[0]
    d = jax.lax.broadcasted_iota(jnp.int32, (DEPTH, HB, W), 0)
    o_ref[0] = (x[None, :, :] == d).astype(jnp.float32)


def kernel(rank, X_in, ones):
    b = X_in.shape[0]
    x = X_in.reshape(b, H, W)
    out = pl.pallas_call(
        _one_hot_kernel,
        grid=(b, H // HB),
        in_specs=[pl.BlockSpec((1, HB, W), lambda i, j: (i, j, 0))],
        out_specs=pl.BlockSpec((1, DEPTH, HB, W), lambda i, j: (i, 0, j, 0)),
        out_shape=jax.ShapeDtypeStruct((b, DEPTH, H, W), jnp.float32),
    )(x)
    return out
